# Initial kernel scaffold; baseline (speedup 1.0000x reference)
import jax
import jax.numpy as jnp
from jax.experimental import pallas as pl


def kernel(x_2, cells_low_repeats_1_2, cells_high_feature_alignment_subcomplex_1_2):
    raise NotImplementedError("write your pallas kernel here")


# speedup vs baseline: 56.4209x; 56.4209x over previous
import functools

import jax
import jax.numpy as jnp
from jax import lax
from jax.experimental import pallas as pl
from jax.experimental.pallas import tpu as pltpu
from jax.experimental.pallas import tpu_sc as plsc

N_HIGH = 10000
D_FEAT = 256
N_OUT = 150000

NC = 2
NS = 16
NW = NC * NS

PW = 4736
STRIDE = 4688
CHUNK = 128
NCHUNK = PW // CHUNK
NVEC = PW // 16
NREPV = N_HIGH // 16
SEARCH_BITS = 14


def _sc_body(x_hbm, rep_hbm, align_hbm, out_hbm,
             rep_v, cum_v, align_v, idx_v, rows_a, rows_b, sem_a, sem_b):
    wid = lax.axis_index("s") * NC + lax.axis_index("c")
    base = pl.multiple_of(jnp.minimum(wid * STRIDE, N_OUT - PW), 16)

    pltpu.sync_copy(rep_hbm, rep_v)
    pltpu.sync_copy(align_hbm.at[pl.ds(base, PW)], align_v)

    def scan_body(k, carry):
        v = rep_v[pl.ds(k * 16, 16)]
        cum_v[pl.ds(k * 16, 16)] = plsc.cumsum(v) + carry
        return carry + jnp.sum(v)

    total = lax.fori_loop(0, NREPV, scan_body, jnp.int32(0))

    def search_body(i, _):
        av = align_v[pl.ds(i * 16, 16)]
        pos = jnp.minimum(jnp.maximum(av, 0), total - 1)
        res = jnp.zeros((16,), jnp.int32)
        for b in range(SEARCH_BITS - 1, -1, -1):
            cand = res + (1 << b)
            cval = plsc.load_gather(cum_v, [jnp.minimum(cand - 1, N_HIGH - 1)])
            take = jnp.logical_and(cand <= N_HIGH, cval <= pos)
            res = jnp.where(take, cand, res)
        idx_v[pl.ds(i * 16, 16)] = res
        return 0

    lax.fori_loop(0, NVEC, search_body, 0)

    def gather_start(c, rows, sem):
        off = pl.multiple_of(c * CHUNK, CHUNK)
        return pltpu.async_copy(x_hbm.at[idx_v.at[pl.ds(off, CHUNK)]], rows, sem)

    def write_out(c, rows):
        off = pl.multiple_of(c * CHUNK, CHUNK)
        pltpu.sync_copy(rows, out_hbm.at[pl.ds(base + off, CHUNK)])

    gather_start(0, rows_a, sem_a).wait()

    def dma_body(c, _):
        nxt = jnp.minimum(c + 1, NCHUNK - 1)
        cp_b = gather_start(nxt, rows_b, sem_b)
        write_out(c, rows_a)
        cp_b.wait()
        nxt2 = jnp.minimum(c + 2, NCHUNK - 1)
        cp_a = gather_start(nxt2, rows_a, sem_a)
        write_out(nxt, rows_b)
        cp_a.wait()
        return 0

    lax.fori_loop(0, (NCHUNK + 1) // 2, lambda p, s: dma_body(p * 2, s), 0)


_mesh = plsc.VectorSubcoreMesh(core_axis_name="c", subcore_axis_name="s")

_sc_kernel = functools.partial(
    pl.kernel,
    out_type=jax.ShapeDtypeStruct((N_OUT, D_FEAT), jnp.float32),
    mesh=_mesh,
    scratch_types=[
        pltpu.VMEM((N_HIGH,), jnp.int32),
        pltpu.VMEM((N_HIGH,), jnp.int32),
        pltpu.VMEM((PW,), jnp.int32),
        pltpu.VMEM((PW,), jnp.int32),
        pltpu.VMEM((CHUNK, D_FEAT), jnp.float32),
        pltpu.VMEM((CHUNK, D_FEAT), jnp.float32),
        pltpu.SemaphoreType.DMA,
        pltpu.SemaphoreType.DMA,
    ],
    compiler_params=pltpu.CompilerParams(needs_layout_passes=False),
)(_sc_body)


@jax.jit
def kernel(x_2, cells_low_repeats_1_2, cells_high_feature_alignment_subcomplex_1_2):
    repeats = cells_low_repeats_1_2.reshape(-1).astype(jnp.int32)
    align = cells_high_feature_alignment_subcomplex_1_2.astype(jnp.int32)
    return _sc_kernel(x_2, repeats, align)
